# baseline (speedup 1.0000x reference)
import functools

import jax
import jax.numpy as jnp
from jax import lax
from jax.experimental import pallas as pl
from jax.experimental.pallas import tpu as pltpu
from jax.experimental.pallas import tpu_sc as plsc

_VOCAB = 100000
_BUCKETS = 100000
_EMB = 64
_BATCH = 4096
_HIST = 20

_NC = 2
_NS = 16
_NW = _NC * _NS
_BPW = _BATCH // _NW
_LANE_CHUNKS = _EMB // 16


def _pool_body(target_hbm, ngrams_t_hbm, emb_hbm, ngram_emb_hbm, out_hbm,
               tidx, nidx, tbuf, gbuf, acc, sem_t, sem_g):
    wid = lax.axis_index("s") * _NC + lax.axis_index("c")
    base = wid * _BPW

    pltpu.sync_copy(target_hbm.at[pl.ds(base, _BPW)], tidx)
    pltpu.sync_copy(ngrams_t_hbm.at[:, pl.ds(base, _BPW)], nidx)

    tcopy = pltpu.async_copy(emb_hbm.at[tidx], tbuf, sem_t)

    def zbody(i, c):
        for k in range(_LANE_CHUNKS):
            acc[i, pl.ds(k * 16, 16)] = jnp.zeros((16,), jnp.float32)
        return c
    lax.fori_loop(0, _BPW, zbody, 0)

    def jbody(j, c):
        pltpu.async_copy(ngram_emb_hbm.at[nidx.at[j]], gbuf, sem_g).wait()

        def ibody(i, c2):
            for k in range(_LANE_CHUNKS):
                s = pl.ds(k * 16, 16)
                plsc.addupdate(acc.at[i, s], gbuf[i, s])
            return c2
        lax.fori_loop(0, _BPW, ibody, 0)
        return c
    lax.fori_loop(0, _HIST, jbody, 0)

    tcopy.wait()

    inv = jnp.float32(1.0 / _HIST)
    def fbody(i, c):
        for k in range(_LANE_CHUNKS):
            s = pl.ds(k * 16, 16)
            acc[i, s] = acc[i, s] * inv + tbuf[i, s]
        return c
    lax.fori_loop(0, _BPW, fbody, 0)

    pltpu.sync_copy(acc, out_hbm.at[pl.ds(base, _BPW)])


_pool = pl.kernel(
    _pool_body,
    out_type=jax.ShapeDtypeStruct((_BATCH, _EMB), jnp.float32),
    mesh=plsc.VectorSubcoreMesh(core_axis_name="c", subcore_axis_name="s",
                                num_cores=_NC, num_subcores=_NS),
    scratch_types=[
        pltpu.VMEM((_BPW,), jnp.int32),
        pltpu.VMEM((_HIST, _BPW), jnp.int32),
        pltpu.VMEM((_BPW, _EMB), jnp.float32),
        pltpu.VMEM((_BPW, _EMB), jnp.float32),
        pltpu.VMEM((_BPW, _EMB), jnp.float32),
        pltpu.SemaphoreType.DMA,
        pltpu.SemaphoreType.DMA,
    ],
    compiler_params=pltpu.CompilerParams(use_tc_tiling_on_sc=False),
)


_BN = 512


def _mm_body(v_ref, w_ref, b_ref, o_ref):
    o_ref[...] = lax.dot_general(
        v_ref[...], w_ref[...], (((1,), (1,)), ((), ())),
        preferred_element_type=jnp.float32) + b_ref[...]


def _project(v, W, b2):
    nv = pl.cdiv(_VOCAB, _BN)
    return pl.pallas_call(
        _mm_body,
        grid=(nv,),
        in_specs=[
            pl.BlockSpec((_BATCH, _EMB), lambda i: (0, 0)),
            pl.BlockSpec((_BN, _EMB), lambda i: (i, 0)),
            pl.BlockSpec((1, _BN), lambda i: (0, i)),
        ],
        out_specs=pl.BlockSpec((_BATCH, _BN), lambda i: (0, i)),
        out_shape=jax.ShapeDtypeStruct((_BATCH, _VOCAB), jnp.float32),
        compiler_params=pltpu.CompilerParams(
            dimension_semantics=("arbitrary",)),
    )(v, W, b2)


def kernel(target, ngrams, emb, ngram_emb, W, b):
    target = target.astype(jnp.int32)
    ngrams_t = ngrams.astype(jnp.int32).T
    v = _pool(target, ngrams_t, emb, ngram_emb)
    return _project(v, W, b.reshape(1, _VOCAB))


# speedup vs baseline: 2.9624x; 2.9624x over previous
import functools

import jax
import jax.numpy as jnp
from jax import lax
from jax.experimental import pallas as pl
from jax.experimental.pallas import tpu as pltpu
from jax.experimental.pallas import tpu_sc as plsc

_VOCAB = 100000
_BUCKETS = 100000
_EMB = 64
_BATCH = 4096
_HIST = 20

_NC = 2
_NS = 16
_NW = _NC * _NS
_BPW = _BATCH // _NW
_LANE_CHUNKS = _EMB // 16


def _pool_body(target_hbm, ngrams_t_hbm, emb_hbm, ngram_emb_hbm, out_hbm,
               tidx, nidx, tbuf, gbuf, acc, sem_t, sem_g):
    wid = lax.axis_index("s") * _NC + lax.axis_index("c")
    base = wid * _BPW

    pltpu.sync_copy(target_hbm.at[pl.ds(base, _BPW)], tidx)
    pltpu.sync_copy(ngrams_t_hbm.at[:, pl.ds(base, _BPW)], nidx)

    tcopy = pltpu.async_copy(emb_hbm.at[tidx], tbuf, sem_t)

    def zbody(i, c):
        for k in range(_LANE_CHUNKS):
            acc[i, pl.ds(k * 16, 16)] = jnp.zeros((16,), jnp.float32)
        return c
    lax.fori_loop(0, _BPW, zbody, 0)

    def jbody(j, c):
        pltpu.async_copy(ngram_emb_hbm.at[nidx.at[j]], gbuf, sem_g).wait()

        def ibody(i, c2):
            for k in range(_LANE_CHUNKS):
                s = pl.ds(k * 16, 16)
                plsc.addupdate(acc.at[i, s], gbuf[i, s])
            return c2
        lax.fori_loop(0, _BPW, ibody, 0)
        return c
    lax.fori_loop(0, _HIST, jbody, 0)

    tcopy.wait()

    inv = jnp.float32(1.0 / _HIST)
    def fbody(i, c):
        for k in range(_LANE_CHUNKS):
            s = pl.ds(k * 16, 16)
            acc[i, s] = acc[i, s] * inv + tbuf[i, s]
        return c
    lax.fori_loop(0, _BPW, fbody, 0)

    pltpu.sync_copy(acc, out_hbm.at[pl.ds(base, _BPW)])


_pool = pl.kernel(
    _pool_body,
    out_type=jax.ShapeDtypeStruct((_BATCH, _EMB), jnp.float32),
    mesh=plsc.VectorSubcoreMesh(core_axis_name="c", subcore_axis_name="s",
                                num_cores=_NC, num_subcores=_NS),
    scratch_types=[
        pltpu.VMEM((_BPW,), jnp.int32),
        pltpu.VMEM((_HIST, _BPW), jnp.int32),
        pltpu.VMEM((_BPW, _EMB), jnp.float32),
        pltpu.VMEM((_BPW, _EMB), jnp.float32),
        pltpu.VMEM((_BPW, _EMB), jnp.float32),
        pltpu.SemaphoreType.DMA,
        pltpu.SemaphoreType.DMA,
    ],
    compiler_params=pltpu.CompilerParams(use_tc_tiling_on_sc=False),
)


_BN = 512


def _mm_body(wt_ref, v_ref, b_ref, o_ref):
    o_ref[...] = lax.dot_general(
        wt_ref[...], v_ref[...], (((0,), (1,)), ((), ())),
        preferred_element_type=jnp.float32) + b_ref[...]


def _project_t(Wt, v, bc):
    nv = pl.cdiv(_VOCAB, _BN)
    return pl.pallas_call(
        _mm_body,
        grid=(nv,),
        in_specs=[
            pl.BlockSpec((_EMB, _BN), lambda i: (0, i)),
            pl.BlockSpec((_BATCH, _EMB), lambda i: (0, 0)),
            pl.BlockSpec((_BN, 1), lambda i: (i, 0)),
        ],
        out_specs=pl.BlockSpec((_BN, _BATCH), lambda i: (i, 0)),
        out_shape=jax.ShapeDtypeStruct((_VOCAB, _BATCH), jnp.float32),
        compiler_params=pltpu.CompilerParams(
            dimension_semantics=("arbitrary",)),
    )(Wt, v, bc)


def kernel(target, ngrams, emb, ngram_emb, W, b):
    target = target.astype(jnp.int32)
    ngrams_t = ngrams.astype(jnp.int32).T
    v = _pool(target, ngrams_t, emb, ngram_emb)
    ot = _project_t(W.T, v, b.reshape(_VOCAB, 1))
    return ot.T


# speedup vs baseline: 3.3703x; 1.1377x over previous
import functools

import jax
import jax.numpy as jnp
from jax import lax
from jax.experimental import pallas as pl
from jax.experimental.pallas import tpu as pltpu
from jax.experimental.pallas import tpu_sc as plsc

_VOCAB = 100000
_BUCKETS = 100000
_EMB = 64
_BATCH = 4096
_HIST = 20

_NC = 2
_NS = 16
_NW = _NC * _NS
_BPW = _BATCH // _NW
_LANE_CHUNKS = _EMB // 16


_DPW = _EMB // _NW
_STRIPE = 512
_NSTRIPE = _BATCH // _STRIPE
_BLK = _STRIPE // 16


def _pool_body(target_hbm, ngrams_t_hbm, emb_t_hbm, ngram_emb_t_hbm, vt_hbm,
               tidx, nidx, plane, acc, sem):
    wid = lax.axis_index("s") * _NC + lax.axis_index("c")
    pltpu.sync_copy(target_hbm, tidx)
    inv = jnp.float32(1.0 / _HIST)

    for d in range(_DPW):
        e = wid * _DPW + d

        pltpu.sync_copy(ngram_emb_t_hbm.at[e], plane)

        def stripe_body(st, c):
            base = st * _STRIPE
            pltpu.sync_copy(ngrams_t_hbm.at[:, pl.ds(base, _STRIPE)], nidx)

            def bblock(bi, c2):
                s = pl.ds(bi * 16, 16)
                part = [plsc.load_gather(plane, [nidx[j, s]]) for j in range(4)]
                for j in range(4, _HIST):
                    part[j % 4] = part[j % 4] + plsc.load_gather(plane, [nidx[j, s]])
                acc[pl.ds(base + bi * 16, 16)] = (part[0] + part[1]) + (part[2] + part[3])
                return c2
            lax.fori_loop(0, _BLK, bblock, 0)
            return c
        lax.fori_loop(0, _NSTRIPE, stripe_body, 0)

        pltpu.sync_copy(emb_t_hbm.at[e], plane)

        def fblock(bi, c):
            s = pl.ds(bi * 16, 16)
            acc[s] = acc[s] * inv + plsc.load_gather(plane, [tidx[s]])
            return c
        lax.fori_loop(0, _BATCH // 16, fblock, 0)

        pltpu.sync_copy(acc, vt_hbm.at[e])


_pool = pl.kernel(
    _pool_body,
    out_type=jax.ShapeDtypeStruct((_EMB, _BATCH), jnp.float32),
    mesh=plsc.VectorSubcoreMesh(core_axis_name="c", subcore_axis_name="s",
                                num_cores=_NC, num_subcores=_NS),
    scratch_types=[
        pltpu.VMEM((_BATCH,), jnp.int32),
        pltpu.VMEM((_HIST, _STRIPE), jnp.int32),
        pltpu.VMEM((_BUCKETS,), jnp.float32),
        pltpu.VMEM((_BATCH,), jnp.float32),
        pltpu.SemaphoreType.DMA,
    ],
    compiler_params=pltpu.CompilerParams(needs_layout_passes=False),
)


_BN = 512


def _mm_body(wt_ref, vt_ref, b_ref, o_ref):
    o_ref[...] = lax.dot_general(
        wt_ref[...], vt_ref[...], (((0,), (0,)), ((), ())),
        preferred_element_type=jnp.float32) + b_ref[...]


def _project_t(Wt, vt, bc):
    nv = pl.cdiv(_VOCAB, _BN)
    return pl.pallas_call(
        _mm_body,
        grid=(nv,),
        in_specs=[
            pl.BlockSpec((_EMB, _BN), lambda i: (0, i)),
            pl.BlockSpec((_EMB, _BATCH), lambda i: (0, 0)),
            pl.BlockSpec((_BN, 1), lambda i: (i, 0)),
        ],
        out_specs=pl.BlockSpec((_BN, _BATCH), lambda i: (i, 0)),
        out_shape=jax.ShapeDtypeStruct((_VOCAB, _BATCH), jnp.float32),
        compiler_params=pltpu.CompilerParams(
            dimension_semantics=("arbitrary",)),
    )(Wt, vt, bc)


def kernel(target, ngrams, emb, ngram_emb, W, b):
    target = target.astype(jnp.int32)
    ngrams_t = ngrams.astype(jnp.int32).T
    vt = _pool(target, ngrams_t, emb.T, ngram_emb.T)
    ot = _project_t(W.T, vt, b.reshape(_VOCAB, 1))
    return ot.T


# speedup vs baseline: 3.4112x; 1.0121x over previous
import functools

import jax
import jax.numpy as jnp
from jax import lax
from jax.experimental import pallas as pl
from jax.experimental.pallas import tpu as pltpu
from jax.experimental.pallas import tpu_sc as plsc

_VOCAB = 100000
_BUCKETS = 100000
_EMB = 64
_BATCH = 4096
_HIST = 20

_NC = 2
_NS = 16
_NW = _NC * _NS
_BPW = _BATCH // _NW
_LANE_CHUNKS = _EMB // 16


_DPW = _EMB // _NW
_STRIPE = 512
_NSTRIPE = _BATCH // _STRIPE
_BLK = _STRIPE // 16


def _pool_body(target_hbm, ngrams_t_hbm, emb_t_hbm, ngram_emb_t_hbm, vt_hbm,
               tidx, nidx, plane, acc, sem):
    wid = lax.axis_index("s") * _NC + lax.axis_index("c")
    pltpu.sync_copy(target_hbm, tidx)
    inv = jnp.float32(1.0 / _HIST)

    for d in range(_DPW):
        e = wid * _DPW + d

        pltpu.sync_copy(ngram_emb_t_hbm.at[e], plane)

        def stripe_body(st, c):
            base = st * _STRIPE
            pltpu.sync_copy(ngrams_t_hbm.at[:, pl.ds(base, _STRIPE)], nidx)

            def bblock(bi, c2):
                s = pl.ds(bi * 16, 16)
                part = [plsc.load_gather(plane, [nidx[j, s]]) for j in range(4)]
                for j in range(4, _HIST):
                    part[j % 4] = part[j % 4] + plsc.load_gather(plane, [nidx[j, s]])
                acc[pl.ds(base + bi * 16, 16)] = (part[0] + part[1]) + (part[2] + part[3])
                return c2
            lax.fori_loop(0, _BLK, bblock, 0)
            return c
        lax.fori_loop(0, _NSTRIPE, stripe_body, 0)

        pltpu.sync_copy(emb_t_hbm.at[e], plane)

        def fblock(bi, c):
            s = pl.ds(bi * 16, 16)
            acc[s] = acc[s] * inv + plsc.load_gather(plane, [tidx[s]])
            return c
        lax.fori_loop(0, _BATCH // 16, fblock, 0)

        pltpu.sync_copy(acc, vt_hbm.at[e])


_pool = pl.kernel(
    _pool_body,
    out_type=jax.ShapeDtypeStruct((_EMB, _BATCH), jnp.float32),
    mesh=plsc.VectorSubcoreMesh(core_axis_name="c", subcore_axis_name="s",
                                num_cores=_NC, num_subcores=_NS),
    scratch_types=[
        pltpu.VMEM((_BATCH,), jnp.int32),
        pltpu.VMEM((_HIST, _STRIPE), jnp.int32),
        pltpu.VMEM((_BUCKETS,), jnp.float32),
        pltpu.VMEM((_BATCH,), jnp.float32),
        pltpu.SemaphoreType.DMA,
    ],
    compiler_params=pltpu.CompilerParams(needs_layout_passes=False),
)


_BN = 1024


def _mm_body(wt_ref, vt_ref, b_ref, o_ref):
    o_ref[...] = lax.dot_general(
        wt_ref[...], vt_ref[...], (((0,), (0,)), ((), ())),
        preferred_element_type=jnp.float32) + b_ref[...]


def _project_t(Wt, vt, bc):
    nv = pl.cdiv(_VOCAB, _BN)
    return pl.pallas_call(
        _mm_body,
        grid=(nv,),
        in_specs=[
            pl.BlockSpec((_EMB, _BN), lambda i: (0, i)),
            pl.BlockSpec((_EMB, _BATCH), lambda i: (0, 0)),
            pl.BlockSpec((_BN, 1), lambda i: (i, 0)),
        ],
        out_specs=pl.BlockSpec((_BN, _BATCH), lambda i: (i, 0)),
        out_shape=jax.ShapeDtypeStruct((_VOCAB, _BATCH), jnp.float32),
        compiler_params=pltpu.CompilerParams(
            dimension_semantics=("arbitrary",)),
    )(Wt, vt, bc)


def kernel(target, ngrams, emb, ngram_emb, W, b):
    target = target.astype(jnp.int32)
    ngrams_t = ngrams.astype(jnp.int32).T
    vt = _pool(target, ngrams_t, emb.T, ngram_emb.T)
    ot = _project_t(W.T, vt, b.reshape(_VOCAB, 1))
    return ot.T
